# baseline (speedup 1.0000x reference)
import jax
import jax.numpy as jnp
from jax.experimental import pallas as pl
from jax.experimental.pallas import tpu as pltpu

_BUCKETS = 256
_TEMP = 0.75
_ITERS = 7
_EPS = 1e-6
_SCALE = 1024.0 ** -0.5
_TILE_B = 8
_NEG = -1e30


def _lse(r, axis):
    m = jnp.max(r, axis=axis, keepdims=True)
    return m + jnp.log(jnp.sum(jnp.exp(r - m), axis=axis, keepdims=True))


def _body(gum_ref, q_ref, k_ref, v_ref, w_ref, o_ref, k2_ref, v2_ref):
    j = pl.program_id(1)
    bsz = q_ref.shape[2]
    d_h = q_ref.shape[3]
    n_tile_rows = _TILE_B * bsz
    n_tile_keys = _TILE_B * 2 * bsz

    @pl.when(j == 0)
    def _routing():
        kf = k_ref[0]
        vf = v_ref[0]
        qf = q_ref[0]
        qs = jnp.sum(qf, axis=1)
        ks = jnp.sum(kf, axis=1)
        x = jnp.concatenate([qs, ks], axis=-1)
        R = jnp.dot(x, w_ref[0], preferred_element_type=jnp.float32)
        R = jnp.where(R >= 0, R, 0.01 * R)
        r = jnp.log(R + _EPS)
        r = (r + gum_ref[0]) / _TEMP
        for _ in range(_ITERS):
            r = r - _lse(r, axis=1)
            r = r - _lse(r, axis=0)
        Rn = jnp.exp(r)
        vals = jnp.max(Rn, axis=1, keepdims=True)
        is_max = Rn == vals
        col = jax.lax.broadcasted_iota(jnp.int32, (_BUCKETS, _BUCKETS), 1)
        first_idx = jnp.min(jnp.where(is_max, col, _BUCKETS),
                            axis=1, keepdims=True)
        R_top = jnp.where(col == first_idx, Rn, 0.0)
        k2d = kf.reshape(_BUCKETS, bsz * d_h)
        v2d = vf.reshape(_BUCKETS, bsz * d_h)
        k_re = jnp.dot(R_top, k2d, preferred_element_type=jnp.float32)
        v_re = jnp.dot(R_top, v2d, preferred_element_type=jnp.float32)
        k2 = jnp.concatenate(
            [k_re.reshape(_BUCKETS, bsz, d_h), kf], axis=1)
        v2 = jnp.concatenate(
            [v_re.reshape(_BUCKETS, bsz, d_h), vf], axis=1)
        k2_ref[...] = k2.reshape(_BUCKETS * 2 * bsz, d_h)
        v2_ref[...] = v2.reshape(_BUCKETS * 2 * bsz, d_h)

    qt = q_ref[0, pl.ds(j * _TILE_B, _TILE_B)].reshape(n_tile_rows, d_h)
    k2t = k2_ref[pl.ds(j * n_tile_keys, n_tile_keys), :]
    v2t = v2_ref[pl.ds(j * n_tile_keys, n_tile_keys), :]
    dots = jax.lax.dot_general(
        qt, k2t, (((1,), (1,)), ((), ())),
        preferred_element_type=jnp.float32) * _SCALE
    row_b = jax.lax.broadcasted_iota(jnp.int32, (n_tile_rows, n_tile_keys), 0) // bsz
    col_b = jax.lax.broadcasted_iota(jnp.int32, (n_tile_rows, n_tile_keys), 1) // (2 * bsz)
    dots = jnp.where(row_b == col_b, dots, _NEG)
    m = jnp.max(dots, axis=1, keepdims=True)
    p = jnp.exp(dots - m)
    p = p / jnp.sum(p, axis=1, keepdims=True)
    outt = jnp.dot(p, v2t, preferred_element_type=jnp.float32)
    o_ref[0] = outt.reshape(_TILE_B, bsz, d_h)


def kernel(q, k, v, sort_linear):
    b, h, t, d_h = q.shape
    bh = b * h
    bsz = t // _BUCKETS
    n_tiles = _BUCKETS // _TILE_B

    qb = q.reshape(bh, _BUCKETS, bsz, d_h)
    kb = k.reshape(bh, _BUCKETS, bsz, d_h)
    vb = v.reshape(bh, _BUCKETS, bsz, d_h)
    W = jnp.broadcast_to(sort_linear, (b, h, 2 * d_h, _BUCKETS)).reshape(
        bh, 2 * d_h, _BUCKETS)
    u_noise = jax.random.uniform(
        jax.random.key(1234), (bh, _BUCKETS, _BUCKETS),
        minval=0.0, maxval=1.0)
    gum = -jnp.log(-jnp.log(u_noise + _EPS) + _EPS)

    out = pl.pallas_call(
        _body,
        grid=(bh, n_tiles),
        in_specs=[
            pl.BlockSpec((1, _BUCKETS, _BUCKETS), lambda i, j: (i, 0, 0)),
            pl.BlockSpec((1, _BUCKETS, bsz, d_h), lambda i, j: (i, 0, 0, 0)),
            pl.BlockSpec((1, _BUCKETS, bsz, d_h), lambda i, j: (i, 0, 0, 0)),
            pl.BlockSpec((1, _BUCKETS, bsz, d_h), lambda i, j: (i, 0, 0, 0)),
            pl.BlockSpec((1, 2 * d_h, _BUCKETS), lambda i, j: (i, 0, 0)),
        ],
        out_specs=pl.BlockSpec(
            (1, _TILE_B, bsz, d_h), lambda i, j: (i, j, 0, 0)),
        out_shape=jax.ShapeDtypeStruct((bh, _BUCKETS, bsz, d_h), jnp.float32),
        scratch_shapes=[
            pltpu.VMEM((_BUCKETS * 2 * bsz, d_h), jnp.float32),
            pltpu.VMEM((_BUCKETS * 2 * bsz, d_h), jnp.float32),
        ],
        compiler_params=pltpu.CompilerParams(
            dimension_semantics=("parallel", "arbitrary"),
        ),
    )(gum, qb, kb, vb, W)
    return out.reshape(b, h, t, d_h)
